# Initial kernel scaffold; baseline (speedup 1.0000x reference)
import jax
import jax.numpy as jnp
from jax.experimental import pallas as pl


def kernel(x):
    raise NotImplementedError("write your pallas kernel here")


# speedup vs baseline: 8.1031x; 8.1031x over previous
import functools

import jax
import jax.numpy as jnp
from jax.experimental import pallas as pl
from jax.experimental.pallas import tpu as pltpu

KEEP = 0.1
ROW_BLOCK = 512


def _body(k, x_ref, o_ref):
    x = x_ref[...]
    bits = jax.lax.bitcast_convert_type(x, jnp.int32) & jnp.int32(0x7FFFFFFF)
    lo = jnp.zeros((x.shape[0], 1), jnp.int32)
    for b in range(30, -1, -1):
        cand = lo | jnp.int32(1 << b)
        cnt = jnp.sum((bits >= cand).astype(jnp.int32), axis=1, keepdims=True)
        lo = jnp.where(cnt >= k, cand, lo)
    thr = jax.lax.bitcast_convert_type(lo, jnp.float32)
    ax = jax.lax.bitcast_convert_type(bits, jnp.float32)
    mask = jax.nn.sigmoid(10.0 * (ax - thr))
    o_ref[...] = x * mask


def kernel(x):
    B, T, D = x.shape
    k = max(1, int(D * KEEP))
    R = B * T
    xr = x.reshape(R, D)
    grid = R // ROW_BLOCK
    out = pl.pallas_call(
        functools.partial(_body, k),
        grid=(grid,),
        in_specs=[pl.BlockSpec((ROW_BLOCK, D), lambda i: (i, 0))],
        out_specs=pl.BlockSpec((ROW_BLOCK, D), lambda i: (i, 0)),
        out_shape=jax.ShapeDtypeStruct((R, D), x.dtype),
    )(xr)
    return out.reshape(B, T, D)


# speedup vs baseline: 12.1142x; 1.4950x over previous
import functools

import jax
import jax.numpy as jnp
from jax.experimental import pallas as pl
from jax.experimental.pallas import tpu as pltpu

KEEP = 0.1
ROW_BLOCK = 512


def _body(k, x_ref, o_ref):
    x = x_ref[...]
    bits = jax.lax.bitcast_convert_type(x, jnp.int32) & jnp.int32(0x7FFFFFFF)
    lo = jnp.zeros((x.shape[0], 1), jnp.int32)
    for b in range(30, 10, -1):
        cand = lo | jnp.int32(1 << b)
        cnt = jnp.sum((bits >= cand).astype(jnp.int32), axis=1, keepdims=True)
        lo = jnp.where(cnt >= k, cand, lo)
    thr = jax.lax.bitcast_convert_type(lo, jnp.float32)
    ax = jax.lax.bitcast_convert_type(bits, jnp.float32)
    mask = jax.nn.sigmoid(10.0 * (ax - thr))
    o_ref[...] = x * mask


def kernel(x):
    B, T, D = x.shape
    k = max(1, int(D * KEEP))
    R = B * T
    xr = x.reshape(R, D)
    grid = R // ROW_BLOCK
    out = pl.pallas_call(
        functools.partial(_body, k),
        grid=(grid,),
        in_specs=[pl.BlockSpec((ROW_BLOCK, D), lambda i: (i, 0))],
        out_specs=pl.BlockSpec((ROW_BLOCK, D), lambda i: (i, 0)),
        out_shape=jax.ShapeDtypeStruct((R, D), x.dtype),
    )(xr)
    return out.reshape(B, T, D)
